# baseline (speedup 1.0000x reference)
import jax
import jax.numpy as jnp
from jax.experimental import pallas as pl
from jax.experimental.pallas import tpu as pltpu

_N_CHUNKS = 8


def _dma_body(in_ref, out_ref, *sems):
    rows = in_ref.shape[0]
    chunk = rows // _N_CHUNKS
    copies = [
        pltpu.make_async_copy(
            in_ref.at[pl.ds(i * chunk, chunk), :],
            out_ref.at[pl.ds(i * chunk, chunk), :],
            sems[i],
        )
        for i in range(_N_CHUNKS)
    ]
    for c in copies:
        c.start()
    for c in copies:
        c.wait()


def kernel(seq_len, matrix):
    del seq_len
    rows, cols = matrix.shape
    return pl.pallas_call(
        _dma_body,
        in_specs=[pl.BlockSpec(memory_space=pltpu.MemorySpace.HBM)],
        out_specs=pl.BlockSpec(memory_space=pltpu.MemorySpace.HBM),
        scratch_shapes=[pltpu.SemaphoreType.DMA] * _N_CHUNKS,
        out_shape=jax.ShapeDtypeStruct((rows, cols), matrix.dtype),
    )(matrix)


# speedup vs baseline: 42.2481x; 42.2481x over previous
import jax
import jax.numpy as jnp
from jax.experimental import pallas as pl
from jax.experimental.pallas import tpu as pltpu

_NBUF = 4
_BLOCK = 512


def _dma_body(in_hbm, out_hbm, *scratch):
    bufs = scratch[:_NBUF]
    rsems = scratch[_NBUF:2 * _NBUF]
    wsems = scratch[2 * _NBUF:3 * _NBUF]
    rows = in_hbm.shape[0]
    nblocks = rows // _BLOCK

    def read(i):
        b = i % _NBUF
        return pltpu.make_async_copy(
            in_hbm.at[pl.ds(i * _BLOCK, _BLOCK), :], bufs[b], rsems[b])

    def write(i):
        b = i % _NBUF
        return pltpu.make_async_copy(
            bufs[b], out_hbm.at[pl.ds(i * _BLOCK, _BLOCK), :], wsems[b])

    for i in range(min(_NBUF, nblocks)):
        read(i).start()
    for i in range(nblocks):
        read(i).wait()
        write(i).start()
        j = i + _NBUF
        if j < nblocks:
            write(i).wait()
            read(j).start()
    for i in range(max(0, nblocks - _NBUF), nblocks):
        write(i).wait()


def kernel(seq_len, matrix):
    del seq_len
    rows, cols = matrix.shape
    return pl.pallas_call(
        _dma_body,
        in_specs=[pl.BlockSpec(memory_space=pltpu.MemorySpace.HBM)],
        out_specs=pl.BlockSpec(memory_space=pltpu.MemorySpace.HBM),
        scratch_shapes=(
            [pltpu.VMEM((_BLOCK, 1024), jnp.float32)] * _NBUF
            + [pltpu.SemaphoreType.DMA] * (2 * _NBUF)
        ),
        out_shape=jax.ShapeDtypeStruct((rows, cols), matrix.dtype),
    )(matrix)


# speedup vs baseline: 48.7309x; 1.1534x over previous
import jax
import jax.numpy as jnp
from jax.experimental import pallas as pl
from jax.experimental.pallas import tpu as pltpu

_NBUF = 4
_BLOCK = 2048


def _dma_body(in_hbm, out_hbm, *scratch):
    bufs = scratch[:_NBUF]
    rsems = scratch[_NBUF:2 * _NBUF]
    wsems = scratch[2 * _NBUF:3 * _NBUF]
    rows = in_hbm.shape[0]
    nblocks = rows // _BLOCK

    def read(i):
        b = i % _NBUF
        return pltpu.make_async_copy(
            in_hbm.at[pl.ds(i * _BLOCK, _BLOCK), :], bufs[b], rsems[b])

    def write(i):
        b = i % _NBUF
        return pltpu.make_async_copy(
            bufs[b], out_hbm.at[pl.ds(i * _BLOCK, _BLOCK), :], wsems[b])

    for i in range(min(_NBUF, nblocks)):
        read(i).start()
    for i in range(nblocks):
        read(i).wait()
        write(i).start()
        j = i + _NBUF
        if j < nblocks:
            write(i).wait()
            read(j).start()
    for i in range(max(0, nblocks - _NBUF), nblocks):
        write(i).wait()


def kernel(seq_len, matrix):
    del seq_len
    rows, cols = matrix.shape
    return pl.pallas_call(
        _dma_body,
        in_specs=[pl.BlockSpec(memory_space=pltpu.MemorySpace.HBM)],
        out_specs=pl.BlockSpec(memory_space=pltpu.MemorySpace.HBM),
        scratch_shapes=(
            [pltpu.VMEM((_BLOCK, 1024), jnp.float32)] * _NBUF
            + [pltpu.SemaphoreType.DMA] * (2 * _NBUF)
        ),
        out_shape=jax.ShapeDtypeStruct((rows, cols), matrix.dtype),
    )(matrix)
